# baseline (speedup 1.0000x reference)
import numpy as np
import jax
import jax.numpy as jnp
from jax.experimental import pallas as pl
from jax.experimental.pallas import tpu as pltpu

NUM_FRAMES = 20
NUM_SAMPLE = 5
DIM = 256
NA = NUM_FRAMES - 1
NF = 1 + NUM_SAMPLE


def _neg_indices():
    rng = np.random.default_rng(0)
    neg = []
    for i in range(NUM_FRAMES - 1):
        keep = np.array(
            [j for j in range(NUM_FRAMES) if j not in (i - 1, i, i + 1)],
            dtype=np.int32,
        )
        ri = np.asarray(rng.permutation(NUM_FRAMES - 3)[:NUM_SAMPLE], dtype=np.int32)
        neg.append(keep[ri])
    return np.stack(neg, axis=0)


NEG_IDX = _neg_indices()

_BP = 128
_NBUF = 6
_MAXW = 64


def _partner(k, t):
    return t + 1 if k == 0 else int(NEG_IDX[t, k - 1])


_EYE = np.eye(NUM_FRAMES, dtype=np.float32)
_R_REP = np.zeros((NUM_FRAMES, 128), dtype=np.float32)
_M_ALL = np.zeros((NUM_FRAMES, 128), dtype=np.float32)
_A_ALL = np.zeros((NUM_FRAMES, 128), dtype=np.float32)
_T_ALL = np.zeros((NUM_FRAMES, 128), dtype=np.float32)
_S6 = np.zeros((128, NA), dtype=np.float32)
for _k in range(NF):
    for _t in range(NUM_FRAMES):
        _R_REP[_t, 20 * _k + _t] = 1.0
    for _t in range(NA):
        _p = _partner(_k, _t)
        _M_ALL[_p, 20 * _k + _t] = 1.0
        _A_ALL[_t, 20 * _k + _t] = 1.0
        _T_ALL[_p, 20 * _k + _t] = 1.0
        _S6[20 * _k + _t, _t] = 1.0


def _chunk_loss(x, eye_ref, rrep_ref, mall_ref, aall_ref, tall_ref, s6_ref,
                seq_b, p0):
    g = jax.lax.dot_general(
        x, x, dimension_numbers=(((2,), (2,)), ((0,), (0,)))
    )

    norms2 = jnp.sum(g * eye_ref[...][None], axis=1)

    grep = jax.lax.dot_general(
        g, rrep_ref[...], dimension_numbers=(((2,), (0,)), ((), ()))
    )
    sel = jnp.sum(grep * mall_ref[...][None], axis=1)

    n2a = jax.lax.dot(norms2, aall_ref[...])
    n2p = jax.lax.dot(norms2, tall_ref[...])
    q = n2a * n2p
    eps2 = jnp.float32(1e-16)
    sims = sel * jax.lax.rsqrt(jnp.maximum(q, eps2))

    es = jnp.exp(sims)
    esum = jax.lax.dot(es, s6_ref[...])
    s0 = sims[:, :NA]
    nll = jnp.log(esum) - s0

    pidx = jax.lax.broadcasted_iota(jnp.int32, (_BP, 1), 0) + p0
    mask = (pidx < seq_b).astype(jnp.float32)
    return jnp.sum(nll * mask) / jnp.float32(NA)


def _body(seq_ref, x_hbm, eye_ref, rrep_ref, mall_ref, aall_ref, tall_ref,
          s6_ref, out_ref, buf, sem, wl, acc):
    nj = x_hbm.shape[1] // _BP

    def _fill(i, mcount):
        b = i // nj
        j = i % nj
        seq_b = seq_ref[b]
        nact = (seq_b + _BP - 1) // _BP

        @pl.when(j < nact)
        def _():
            wl[0, mcount] = b
            wl[1, mcount] = j

        return mcount + jnp.where(j < nact, 1, 0)

    m = jax.lax.fori_loop(0, 4 * nj, _fill, jnp.int32(0))
    acc[0] = jnp.float32(0.0)

    def _copy(i, slot):
        b = wl[0, i]
        j = wl[1, i]
        return pltpu.make_async_copy(
            x_hbm.at[b, pl.ds(j * _BP, _BP)], buf.at[slot], sem.at[slot]
        )

    for s in range(_NBUF):
        @pl.when(s < m)
        def _(s=s):
            _copy(s, s).start()

    def _step(it, _):
        for s in range(_NBUF):
            i = _NBUF * it + s

            @pl.when(i < m)
            def _(i=i, s=s):
                _copy(i, s).wait()
                sb = seq_ref[wl[0, i]]
                acc[0] += _chunk_loss(
                    buf[s], eye_ref, rrep_ref, mall_ref, aall_ref, tall_ref,
                    s6_ref, sb, wl[1, i] * _BP,
                )

                @pl.when(i + _NBUF < m)
                def _():
                    _copy(i + _NBUF, s).start()

        return 0

    jax.lax.fori_loop(0, (m + _NBUF - 1) // _NBUF, _step, 0)

    num_seg = (seq_ref[0] + seq_ref[1] + seq_ref[2] + seq_ref[3]).astype(jnp.float32)
    out_ref[0, 0] = acc[0] / num_seg


def kernel(output, seq_len):
    B, P, F, D = output.shape
    out = pl.pallas_call(
        _body,
        grid=(),
        in_specs=[
            pl.BlockSpec(memory_space=pltpu.SMEM),
            pl.BlockSpec(memory_space=pl.ANY),
            pl.BlockSpec(memory_space=pltpu.VMEM),
            pl.BlockSpec(memory_space=pltpu.VMEM),
            pl.BlockSpec(memory_space=pltpu.VMEM),
            pl.BlockSpec(memory_space=pltpu.VMEM),
            pl.BlockSpec(memory_space=pltpu.VMEM),
            pl.BlockSpec(memory_space=pltpu.VMEM),
        ],
        out_specs=pl.BlockSpec(memory_space=pltpu.SMEM),
        out_shape=jax.ShapeDtypeStruct((1, 1), jnp.float32),
        scratch_shapes=[
            pltpu.VMEM((_NBUF, _BP, F, D), jnp.float32),
            pltpu.SemaphoreType.DMA((_NBUF,)),
            pltpu.SMEM((2, _MAXW), jnp.int32),
            pltpu.SMEM((1,), jnp.float32),
        ],
    )(seq_len, output, _EYE, _R_REP, _M_ALL, _A_ALL, _T_ALL, _S6)
    return out[0, 0]


# speedup vs baseline: 1.0273x; 1.0273x over previous
import numpy as np
import jax
import jax.numpy as jnp
from jax.experimental import pallas as pl
from jax.experimental.pallas import tpu as pltpu

NUM_FRAMES = 20
NUM_SAMPLE = 5
DIM = 256
NA = NUM_FRAMES - 1
NF = 1 + NUM_SAMPLE


def _neg_indices():
    rng = np.random.default_rng(0)
    neg = []
    for i in range(NUM_FRAMES - 1):
        keep = np.array(
            [j for j in range(NUM_FRAMES) if j not in (i - 1, i, i + 1)],
            dtype=np.int32,
        )
        ri = np.asarray(rng.permutation(NUM_FRAMES - 3)[:NUM_SAMPLE], dtype=np.int32)
        neg.append(keep[ri])
    return np.stack(neg, axis=0)


NEG_IDX = _neg_indices()

_BP = 256
_NBUF = 4
_MAXW = 64


def _partner(k, t):
    return t + 1 if k == 0 else int(NEG_IDX[t, k - 1])


_EYE = np.eye(NUM_FRAMES, dtype=np.float32)
_R_REP = np.zeros((NUM_FRAMES, 128), dtype=np.float32)
_M_ALL = np.zeros((NUM_FRAMES, 128), dtype=np.float32)
_A_ALL = np.zeros((NUM_FRAMES, 128), dtype=np.float32)
_T_ALL = np.zeros((NUM_FRAMES, 128), dtype=np.float32)
_S6 = np.zeros((128, NA), dtype=np.float32)
for _k in range(NF):
    for _t in range(NUM_FRAMES):
        _R_REP[_t, 20 * _k + _t] = 1.0
    for _t in range(NA):
        _p = _partner(_k, _t)
        _M_ALL[_p, 20 * _k + _t] = 1.0
        _A_ALL[_t, 20 * _k + _t] = 1.0
        _T_ALL[_p, 20 * _k + _t] = 1.0
        _S6[20 * _k + _t, _t] = 1.0


def _chunk_loss(x, eye_ref, rrep_ref, mall_ref, aall_ref, tall_ref, s6_ref,
                seq_b, p0):
    g = jax.lax.dot_general(
        x, x, dimension_numbers=(((2,), (2,)), ((0,), (0,))),
        precision=jax.lax.Precision.DEFAULT,
    )

    norms2 = jnp.sum(g * eye_ref[...][None], axis=1)

    grep = jax.lax.dot_general(
        g, rrep_ref[...], dimension_numbers=(((2,), (0,)), ((), ()))
    )
    sel = jnp.sum(grep * mall_ref[...][None], axis=1)

    n2a = jax.lax.dot(norms2, aall_ref[...])
    n2p = jax.lax.dot(norms2, tall_ref[...])
    q = n2a * n2p
    eps2 = jnp.float32(1e-16)
    sims = sel * jax.lax.rsqrt(jnp.maximum(q, eps2))

    es = jnp.exp(sims)
    esum = jax.lax.dot(es, s6_ref[...])
    s0 = sims[:, :NA]
    nll = jnp.log(esum) - s0

    pidx = jax.lax.broadcasted_iota(jnp.int32, (_BP, 1), 0) + p0
    mask = (pidx < seq_b).astype(jnp.float32)
    return jnp.sum(nll * mask) / jnp.float32(NA)


def _body(seq_ref, x_hbm, eye_ref, rrep_ref, mall_ref, aall_ref, tall_ref,
          s6_ref, out_ref, buf, sem, wl, acc):
    nj = x_hbm.shape[1] // _BP

    def _fill(i, mcount):
        b = i // nj
        j = i % nj
        seq_b = seq_ref[b]
        nact = (seq_b + _BP - 1) // _BP

        @pl.when(j < nact)
        def _():
            wl[0, mcount] = b
            wl[1, mcount] = j

        return mcount + jnp.where(j < nact, 1, 0)

    m = jax.lax.fori_loop(0, 4 * nj, _fill, jnp.int32(0))
    acc[0] = jnp.float32(0.0)

    def _copy(i, slot):
        b = wl[0, i]
        j = wl[1, i]
        return pltpu.make_async_copy(
            x_hbm.at[b, pl.ds(j * _BP, _BP)], buf.at[slot], sem.at[slot]
        )

    for s in range(_NBUF):
        @pl.when(s < m)
        def _(s=s):
            _copy(s, s).start()

    def _step(it, _):
        for s in range(_NBUF):
            i = _NBUF * it + s

            @pl.when(i < m)
            def _(i=i, s=s):
                _copy(i, s).wait()
                sb = seq_ref[wl[0, i]]
                acc[0] += _chunk_loss(
                    buf[s], eye_ref, rrep_ref, mall_ref, aall_ref, tall_ref,
                    s6_ref, sb, wl[1, i] * _BP,
                )

                @pl.when(i + _NBUF < m)
                def _():
                    _copy(i + _NBUF, s).start()

        return 0

    jax.lax.fori_loop(0, (m + _NBUF - 1) // _NBUF, _step, 0)

    num_seg = (seq_ref[0] + seq_ref[1] + seq_ref[2] + seq_ref[3]).astype(jnp.float32)
    out_ref[0, 0] = acc[0] / num_seg


def kernel(output, seq_len):
    B, P, F, D = output.shape
    out = pl.pallas_call(
        _body,
        grid=(),
        in_specs=[
            pl.BlockSpec(memory_space=pltpu.SMEM),
            pl.BlockSpec(memory_space=pl.ANY),
            pl.BlockSpec(memory_space=pltpu.VMEM),
            pl.BlockSpec(memory_space=pltpu.VMEM),
            pl.BlockSpec(memory_space=pltpu.VMEM),
            pl.BlockSpec(memory_space=pltpu.VMEM),
            pl.BlockSpec(memory_space=pltpu.VMEM),
            pl.BlockSpec(memory_space=pltpu.VMEM),
        ],
        out_specs=pl.BlockSpec(memory_space=pltpu.SMEM),
        out_shape=jax.ShapeDtypeStruct((1, 1), jnp.float32),
        scratch_shapes=[
            pltpu.VMEM((_NBUF, _BP, F, D), jnp.float32),
            pltpu.SemaphoreType.DMA((_NBUF,)),
            pltpu.SMEM((2, _MAXW), jnp.int32),
            pltpu.SMEM((1,), jnp.float32),
        ],
    )(seq_len, output, _EYE, _R_REP, _M_ALL, _A_ALL, _T_ALL, _S6)
    return out[0, 0]
